# baseline (speedup 1.0000x reference)
import jax
import jax.numpy as jnp
from jax import lax
from jax.experimental import pallas as pl
from jax.experimental.pallas import tpu as pltpu
from jax.experimental.pallas import tpu_sc as plsc

VOCAB = 1000000
SPLIT = 512000
ROWS = 4096
SEQ = 26
EMBED = 64
NC = 2
NS = 16
NW = NC * NS

N_LOOK = ROWS * SEQ
D_PER_W = 2 * N_LOOK // NW
I_PER_CHUNK = 104
NCHUNK = D_PER_W // I_PER_CHUNK
NBUF = 4

TP_BLK = 1024
TP_GRID = SPLIT // TP_BLK
TP_HI_MAX = (VOCAB - 1) // TP_BLK


def _tp_body(lo_ref, hi_ref, o_ref):
    o_ref[...] = jnp.concatenate([lo_ref[...].T, hi_ref[...].T], axis=1)


def _transpose_table(tt):
    return pl.pallas_call(
        _tp_body,
        grid=(TP_GRID,),
        in_specs=[
            pl.BlockSpec((EMBED, TP_BLK), lambda i: (0, i)),
            pl.BlockSpec(
                (EMBED, TP_BLK),
                lambda i: (0, jnp.minimum(i + TP_GRID, TP_HI_MAX))),
        ],
        out_specs=pl.BlockSpec((TP_BLK, 2 * EMBED), lambda i: (i, 0)),
        out_shape=jax.ShapeDtypeStruct((SPLIT, 2 * EMBED), jnp.float32),
    )(tt, tt)


def _emb_body(didx_hbm, t2_hbm, out_hbm, idx_v, rows_v, *sems):
    wid = lax.axis_index("s") * NC + lax.axis_index("c")
    gsems = sems[:NBUF]
    osems = sems[NBUF:]
    o0 = wid * D_PER_W
    pltpu.sync_copy(didx_hbm.at[pl.ds(o0, D_PER_W)], idx_v)

    def g_desc(i, b):
        return pltpu.make_async_copy(
            t2_hbm.at[idx_v.at[pl.ds(i * I_PER_CHUNK, I_PER_CHUNK)]],
            rows_v.at[b], gsems[b])

    def o_desc(i, b):
        return pltpu.make_async_copy(
            rows_v.at[b],
            out_hbm.at[pl.ds(o0 + i * I_PER_CHUNK, I_PER_CHUNK)],
            osems[b])

    for b in range(NBUF):
        g_desc(b, b).start()

    @pl.loop(0, NCHUNK, step=NBUF)
    def _round(j0):
        for b in range(NBUF):
            i = j0 + b
            g_desc(i, b).wait()
            o_desc(i, b).start()

            @pl.when(i + NBUF < NCHUNK)
            def _refill():
                o_desc(i, b).wait()
                g_desc(i + NBUF, b).start()

    for b in range(NBUF):
        o_desc(NCHUNK - NBUF + b, b).wait()


def _gather(didx, t2_flat):
    k = pl.kernel(
        _emb_body,
        mesh=plsc.VectorSubcoreMesh(core_axis_name="c", subcore_axis_name="s"),
        out_type=jax.ShapeDtypeStruct((2 * N_LOOK, EMBED // 2), jnp.float32),
        scratch_types=[
            pltpu.VMEM((D_PER_W,), jnp.int32),
            pltpu.VMEM((NBUF, I_PER_CHUNK, EMBED // 2), jnp.float32),
        ] + [pltpu.SemaphoreType.DMA] * (2 * NBUF),
        compiler_params=pltpu.CompilerParams(use_tc_tiling_on_sc=False),
    )
    return k(didx, t2_flat)


@jax.jit
def _run(x, table):
    xf = x.reshape(-1).astype(jnp.int32)
    base = jnp.where(xf < SPLIT, 4 * xf, 4 * (xf - SPLIT) + 2)
    didx = jnp.stack([base, base + 1], axis=-1).reshape(-1)
    t2 = _transpose_table(table.T)
    out = _gather(didx, t2.reshape(4 * SPLIT, EMBED // 2))
    return out.reshape(ROWS, SEQ, EMBED)


def kernel(x, table):
    return _run(x, table)


# speedup vs baseline: 1.4268x; 1.4268x over previous
import jax
import jax.numpy as jnp
from jax import lax
from jax.experimental import pallas as pl
from jax.experimental.pallas import tpu as pltpu
from jax.experimental.pallas import tpu_sc as plsc

VOCAB = 1000000
SPLIT = 512000
ROWS = 4096
SEQ = 26
EMBED = 64
NC = 2
NS = 16
NW = NC * NS

N_LOOK = ROWS * SEQ
L_PER_W = N_LOOK // NW
I_PER_CHUNK = 104
NCHUNK = L_PER_W // I_PER_CHUNK
NBUF = 4

TP_BLK = 2048
TP_GRID = SPLIT // TP_BLK
TP_HI_MAX = (VOCAB - 1) // TP_BLK


def _tp_body(lo_ref, hi_ref, o_ref):
    o_ref[...] = jnp.concatenate([lo_ref[...].T, hi_ref[...].T], axis=1)


def _transpose_table(tt):
    return pl.pallas_call(
        _tp_body,
        grid=(TP_GRID,),
        in_specs=[
            pl.BlockSpec((EMBED, TP_BLK), lambda i: (0, i)),
            pl.BlockSpec(
                (EMBED, TP_BLK),
                lambda i: (0, jnp.minimum(i + TP_GRID, TP_HI_MAX))),
        ],
        out_specs=pl.BlockSpec((TP_BLK, 2 * EMBED), lambda i: (i, 0)),
        out_shape=jax.ShapeDtypeStruct((SPLIT, 2 * EMBED), jnp.float32),
    )(tt, tt)


def _emb_body(sidx_hbm, t2_hbm, out_hbm, idx_v, rows_v, *sems):
    wid = lax.axis_index("s") * NC + lax.axis_index("c")
    gsems = sems[:NBUF]
    osems = sems[NBUF:]
    pltpu.sync_copy(sidx_hbm.at[pl.ds(wid * L_PER_W, L_PER_W)], idx_v)

    def g_desc(i, b):
        return pltpu.make_async_copy(
            t2_hbm.at[idx_v.at[pl.ds(i * I_PER_CHUNK, I_PER_CHUNK)]],
            rows_v.at[b], gsems[b])

    def o_desc(i, b):
        return pltpu.make_async_copy(
            rows_v.at[b], out_hbm.at[wid * NCHUNK + i], osems[b])

    for b in range(NBUF):
        g_desc(b, b).start()

    @pl.loop(0, NCHUNK, step=NBUF)
    def _round(j0):
        for b in range(NBUF):
            i = j0 + b
            g_desc(i, b).wait()
            o_desc(i, b).start()

            @pl.when(i + NBUF < NCHUNK)
            def _refill():
                o_desc(i, b).wait()
                g_desc(i + NBUF, b).start()

    for b in range(NBUF):
        o_desc(NCHUNK - NBUF + b, b).wait()


def _gather(sidx, t2_flat):
    k = pl.kernel(
        _emb_body,
        mesh=plsc.VectorSubcoreMesh(core_axis_name="c", subcore_axis_name="s"),
        out_type=jax.ShapeDtypeStruct(
            (N_LOOK // I_PER_CHUNK, I_PER_CHUNK, EMBED), jnp.float32),
        scratch_types=[
            pltpu.VMEM((L_PER_W,), jnp.int32),
            pltpu.VMEM((NBUF, I_PER_CHUNK, EMBED), jnp.float32),
        ] + [pltpu.SemaphoreType.DMA] * (2 * NBUF),
        compiler_params=pltpu.CompilerParams(use_tc_tiling_on_sc=False),
    )
    return k(sidx, t2_flat)


@jax.jit
def _run(x, table):
    xf = x.reshape(-1).astype(jnp.int32)
    sidx = jnp.where(xf < SPLIT, 2 * xf, 2 * (xf - SPLIT) + 1)
    t2 = _transpose_table(table.T)
    out = _gather(sidx, t2.reshape(2 * SPLIT, EMBED))
    return out.reshape(ROWS, SEQ, EMBED)


def kernel(x, table):
    return _run(x, table)


# speedup vs baseline: 1.7009x; 1.1921x over previous
import jax
import jax.numpy as jnp
from jax import lax
from jax.experimental import pallas as pl
from jax.experimental.pallas import tpu as pltpu
from jax.experimental.pallas import tpu_sc as plsc

VOCAB = 1000000
SPLIT = 512000
ROWS = 4096
SEQ = 26
EMBED = 64
NC = 2
NS = 16
NW = NC * NS

N_LOOK = ROWS * SEQ
L_PER_W = N_LOOK // NW
I_PER_CHUNK = 104
NCHUNK = L_PER_W // I_PER_CHUNK
NBUF = 4

TP_BLK = 4096
TP_GRID = SPLIT // TP_BLK
TP_HI_MAX = (VOCAB - 1) // TP_BLK


def _tp_body(lo_ref, hi_ref, o_ref):
    o_ref[...] = jnp.concatenate([lo_ref[...].T, hi_ref[...].T], axis=1)


def _transpose_table(tt):
    return pl.pallas_call(
        _tp_body,
        grid=(TP_GRID,),
        in_specs=[
            pl.BlockSpec((EMBED, TP_BLK), lambda i: (0, i)),
            pl.BlockSpec(
                (EMBED, TP_BLK),
                lambda i: (0, jnp.minimum(i + TP_GRID, TP_HI_MAX))),
        ],
        out_specs=pl.BlockSpec((TP_BLK, 2 * EMBED), lambda i: (i, 0)),
        out_shape=jax.ShapeDtypeStruct((SPLIT, 2 * EMBED), jnp.float32),
    )(tt, tt)


def _emb_body(sidx_hbm, t2_hbm, out_hbm, idx_v, rows_v, *sems):
    wid = lax.axis_index("s") * NC + lax.axis_index("c")
    gsems = sems[:NBUF]
    osems = sems[NBUF:]
    pltpu.sync_copy(sidx_hbm.at[pl.ds(wid * L_PER_W, L_PER_W)], idx_v)

    def g_desc(i, b):
        return pltpu.make_async_copy(
            t2_hbm.at[idx_v.at[pl.ds(i * I_PER_CHUNK, I_PER_CHUNK)]],
            rows_v.at[b], gsems[b])

    def o_desc(i, b):
        return pltpu.make_async_copy(
            rows_v.at[b], out_hbm.at[wid * NCHUNK + i], osems[b])

    for b in range(NBUF):
        g_desc(b, b).start()

    @pl.loop(0, NCHUNK, step=NBUF)
    def _round(j0):
        for b in range(NBUF):
            i = j0 + b
            g_desc(i, b).wait()
            o_desc(i, b).start()

            @pl.when(i + NBUF < NCHUNK)
            def _refill():
                o_desc(i, b).wait()
                g_desc(i + NBUF, b).start()

    for b in range(NBUF):
        o_desc(NCHUNK - NBUF + b, b).wait()


def _gather(sidx, t2_flat):
    k = pl.kernel(
        _emb_body,
        mesh=plsc.VectorSubcoreMesh(core_axis_name="c", subcore_axis_name="s"),
        out_type=jax.ShapeDtypeStruct(
            (N_LOOK // I_PER_CHUNK, I_PER_CHUNK, EMBED), jnp.float32),
        scratch_types=[
            pltpu.VMEM((L_PER_W,), jnp.int32),
            pltpu.VMEM((NBUF, I_PER_CHUNK, EMBED), jnp.float32),
        ] + [pltpu.SemaphoreType.DMA] * (2 * NBUF),
        compiler_params=pltpu.CompilerParams(use_tc_tiling_on_sc=False),
    )
    return k(sidx, t2_flat)


@jax.jit
def _run(x, table):
    xf = x.reshape(-1).astype(jnp.int32)
    sidx = jnp.where(xf < SPLIT, 2 * xf, 2 * (xf - SPLIT) + 1)
    t2 = _transpose_table(table.T)
    out = _gather(sidx, t2.reshape(2 * SPLIT, EMBED))
    return out.reshape(ROWS, SEQ, EMBED)


def kernel(x, table):
    return _run(x, table)


# speedup vs baseline: 1.8157x; 1.0675x over previous
import jax
import jax.numpy as jnp
from jax import lax
from jax.experimental import pallas as pl
from jax.experimental.pallas import tpu as pltpu
from jax.experimental.pallas import tpu_sc as plsc

VOCAB = 1000000
SPLIT = 512000
ROWS = 4096
SEQ = 26
EMBED = 64
NC = 2
NS = 16
NW = NC * NS

N_LOOK = ROWS * SEQ
L_PER_W = N_LOOK // NW
I_PER_CHUNK = 104
NCHUNK = L_PER_W // I_PER_CHUNK
NBUF = 4

TP_BLK = 6400
TP_GRID = SPLIT // TP_BLK
TP_HI_MAX = (VOCAB - 1) // TP_BLK


def _tp_body(lo_ref, hi_ref, o_ref):
    o_ref[...] = jnp.concatenate([lo_ref[...].T, hi_ref[...].T], axis=1)


def _transpose_table(tt):
    return pl.pallas_call(
        _tp_body,
        grid=(TP_GRID,),
        in_specs=[
            pl.BlockSpec((EMBED, TP_BLK), lambda i: (0, i)),
            pl.BlockSpec(
                (EMBED, TP_BLK),
                lambda i: (0, jnp.minimum(i + TP_GRID, TP_HI_MAX))),
        ],
        out_specs=pl.BlockSpec((TP_BLK, 2 * EMBED), lambda i: (i, 0)),
        out_shape=jax.ShapeDtypeStruct((SPLIT, 2 * EMBED), jnp.float32),
    )(tt, tt)


def _emb_body(sidx_hbm, t2_hbm, out_hbm, idx_v, rows_v, *sems):
    wid = lax.axis_index("s") * NC + lax.axis_index("c")
    gsems = sems[:NBUF]
    osems = sems[NBUF:]
    pltpu.sync_copy(sidx_hbm.at[pl.ds(wid * L_PER_W, L_PER_W)], idx_v)

    def g_desc(i, b):
        return pltpu.make_async_copy(
            t2_hbm.at[idx_v.at[pl.ds(i * I_PER_CHUNK, I_PER_CHUNK)]],
            rows_v.at[b], gsems[b])

    def o_desc(i, b):
        return pltpu.make_async_copy(
            rows_v.at[b], out_hbm.at[wid * NCHUNK + i], osems[b])

    for b in range(NBUF):
        g_desc(b, b).start()

    @pl.loop(0, NCHUNK, step=NBUF)
    def _round(j0):
        for b in range(NBUF):
            i = j0 + b
            g_desc(i, b).wait()
            o_desc(i, b).start()

            @pl.when(i + NBUF < NCHUNK)
            def _refill():
                o_desc(i, b).wait()
                g_desc(i + NBUF, b).start()

    for b in range(NBUF):
        o_desc(NCHUNK - NBUF + b, b).wait()


def _gather(sidx, t2_flat):
    k = pl.kernel(
        _emb_body,
        mesh=plsc.VectorSubcoreMesh(core_axis_name="c", subcore_axis_name="s"),
        out_type=jax.ShapeDtypeStruct(
            (N_LOOK // I_PER_CHUNK, I_PER_CHUNK, EMBED), jnp.float32),
        scratch_types=[
            pltpu.VMEM((L_PER_W,), jnp.int32),
            pltpu.VMEM((NBUF, I_PER_CHUNK, EMBED), jnp.float32),
        ] + [pltpu.SemaphoreType.DMA] * (2 * NBUF),
        compiler_params=pltpu.CompilerParams(use_tc_tiling_on_sc=False),
    )
    return k(sidx, t2_flat)


@jax.jit
def _run(x, table):
    xf = x.reshape(-1).astype(jnp.int32)
    sidx = jnp.where(xf < SPLIT, 2 * xf, 2 * (xf - SPLIT) + 1)
    t2 = _transpose_table(table.T)
    out = _gather(sidx, t2.reshape(2 * SPLIT, EMBED))
    return out.reshape(ROWS, SEQ, EMBED)


def kernel(x, table):
    return _run(x, table)


# speedup vs baseline: 1.9335x; 1.0649x over previous
import jax
import jax.numpy as jnp
from jax import lax
from jax.experimental import pallas as pl
from jax.experimental.pallas import tpu as pltpu
from jax.experimental.pallas import tpu_sc as plsc

VOCAB = 1000000
SPLIT = 512000
ROWS = 4096
SEQ = 26
EMBED = 64
NC = 2
NS = 16
NW = NC * NS

N_LOOK = ROWS * SEQ
L_PER_W = N_LOOK // NW
I_PER_CHUNK = 104
NCHUNK = L_PER_W // I_PER_CHUNK
NBUF = 4

TP_BLK = 12800
TP_GRID = SPLIT // TP_BLK
TP_HI_MAX = (VOCAB - 1) // TP_BLK


def _tp_body(lo_ref, hi_ref, o_ref):
    o_ref[...] = jnp.concatenate([lo_ref[...].T, hi_ref[...].T], axis=1)


def _transpose_table(tt):
    return pl.pallas_call(
        _tp_body,
        grid=(TP_GRID,),
        in_specs=[
            pl.BlockSpec((EMBED, TP_BLK), lambda i: (0, i)),
            pl.BlockSpec(
                (EMBED, TP_BLK),
                lambda i: (0, jnp.minimum(i + TP_GRID, TP_HI_MAX))),
        ],
        out_specs=pl.BlockSpec((TP_BLK, 2 * EMBED), lambda i: (i, 0)),
        out_shape=jax.ShapeDtypeStruct((SPLIT, 2 * EMBED), jnp.float32),
    )(tt, tt)


def _emb_body(sidx_hbm, t2_hbm, out_hbm, idx_v, rows_v, *sems):
    wid = lax.axis_index("s") * NC + lax.axis_index("c")
    gsems = sems[:NBUF]
    osems = sems[NBUF:]
    pltpu.sync_copy(sidx_hbm.at[pl.ds(wid * L_PER_W, L_PER_W)], idx_v)

    def g_desc(i, b):
        return pltpu.make_async_copy(
            t2_hbm.at[idx_v.at[pl.ds(i * I_PER_CHUNK, I_PER_CHUNK)]],
            rows_v.at[b], gsems[b])

    def o_desc(i, b):
        return pltpu.make_async_copy(
            rows_v.at[b], out_hbm.at[wid * NCHUNK + i], osems[b])

    for b in range(NBUF):
        g_desc(b, b).start()

    @pl.loop(0, NCHUNK, step=NBUF)
    def _round(j0):
        for b in range(NBUF):
            i = j0 + b
            g_desc(i, b).wait()
            o_desc(i, b).start()

            @pl.when(i + NBUF < NCHUNK)
            def _refill():
                o_desc(i, b).wait()
                g_desc(i + NBUF, b).start()

    for b in range(NBUF):
        o_desc(NCHUNK - NBUF + b, b).wait()


def _gather(sidx, t2_flat):
    k = pl.kernel(
        _emb_body,
        mesh=plsc.VectorSubcoreMesh(core_axis_name="c", subcore_axis_name="s"),
        out_type=jax.ShapeDtypeStruct(
            (N_LOOK // I_PER_CHUNK, I_PER_CHUNK, EMBED), jnp.float32),
        scratch_types=[
            pltpu.VMEM((L_PER_W,), jnp.int32),
            pltpu.VMEM((NBUF, I_PER_CHUNK, EMBED), jnp.float32),
        ] + [pltpu.SemaphoreType.DMA] * (2 * NBUF),
        compiler_params=pltpu.CompilerParams(use_tc_tiling_on_sc=False),
    )
    return k(sidx, t2_flat)


@jax.jit
def _run(x, table):
    xf = x.reshape(-1).astype(jnp.int32)
    sidx = jnp.where(xf < SPLIT, 2 * xf, 2 * (xf - SPLIT) + 1)
    t2 = _transpose_table(table.T)
    out = _gather(sidx, t2.reshape(2 * SPLIT, EMBED))
    return out.reshape(ROWS, SEQ, EMBED)


def kernel(x, table):
    return _run(x, table)
